# Initial kernel scaffold; baseline (speedup 1.0000x reference)
import functools

import jax
import jax.numpy as jnp
from jax import lax
from jax.experimental import pallas as pl
from jax.experimental.pallas import tpu as pltpu
from jax.experimental.pallas import tpu_sc as plsc

D = 64
NW = 32
CHUNK = 128


@functools.partial(jax.jit, static_argnums=(2, 3))
def _sc_embedding_gather(table, idx2d, b_per_w, n_chunks):
    B = b_per_w * NW
    mesh = plsc.VectorSubcoreMesh(core_axis_name="c", subcore_axis_name="s")

    @functools.partial(
        pl.kernel,
        mesh=mesh,
        out_type=jax.ShapeDtypeStruct((B, D), jnp.float32),
        scratch_types=[
            pltpu.VMEM((n_chunks, CHUNK), jnp.int32),
            pltpu.VMEM((CHUNK, D), jnp.float32),
            pltpu.SemaphoreType.DMA,
        ],
    )
    def k(table_hbm, idx_hbm, out_hbm, idx_v, rows_v, sem):
        wid = lax.axis_index("s") * 2 + lax.axis_index("c")
        base = wid * b_per_w
        pltpu.sync_copy(idx_hbm.at[pl.ds(wid * n_chunks, n_chunks)], idx_v)

        def body(j, carry):
            pltpu.async_copy(table_hbm.at[idx_v.at[j]], rows_v, sem).wait()
            pltpu.sync_copy(rows_v, out_hbm.at[pl.ds(base + j * CHUNK, CHUNK)])
            return carry

        lax.fori_loop(0, n_chunks, body, 0)

    return k(table, idx2d)


def kernel(x, x_lengths, x_max_length, batch_size, word_embeddings):
    bsz, seq = x.shape
    B = bsz * seq
    b_per_w = B // NW
    n_chunks = b_per_w // CHUNK
    idx2d = x.reshape(NW * n_chunks, CHUNK).astype(jnp.int32)
    out = _sc_embedding_gather(word_embeddings, idx2d, b_per_w, n_chunks)
    return out.reshape(bsz, seq, D)


# speedup vs baseline: 4.0808x; 4.0808x over previous
import functools

import jax
import jax.numpy as jnp
from jax import lax
from jax.experimental import pallas as pl
from jax.experimental.pallas import tpu as pltpu
from jax.experimental.pallas import tpu_sc as plsc

D = 64
NW = 32
CHUNK = 128


@functools.partial(jax.jit, static_argnums=(2, 3))
def _sc_embedding_gather(table, idx_flat, b_per_w, n_chunks):
    B = b_per_w * NW
    mesh = plsc.VectorSubcoreMesh(core_axis_name="c", subcore_axis_name="s")

    @functools.partial(
        pl.kernel,
        mesh=mesh,
        compiler_params=pltpu.CompilerParams(use_tc_tiling_on_sc=False),
        out_type=jax.ShapeDtypeStruct((B, D), jnp.float32),
        scratch_types=[
            pltpu.VMEM((b_per_w,), jnp.int32),
            pltpu.VMEM((CHUNK, D), jnp.float32),
            pltpu.SemaphoreType.DMA,
        ],
    )
    def k(table_hbm, idx_hbm, out_hbm, idx_v, rows_v, sem):
        wid = lax.axis_index("s") * 2 + lax.axis_index("c")
        base = wid * b_per_w
        pltpu.sync_copy(idx_hbm.at[pl.ds(base, b_per_w)], idx_v)

        def body(j, carry):
            pltpu.async_copy(
                table_hbm.at[idx_v.at[pl.ds(j * CHUNK, CHUNK)]], rows_v, sem
            ).wait()
            pltpu.sync_copy(rows_v, out_hbm.at[pl.ds(base + j * CHUNK, CHUNK)])
            return carry

        lax.fori_loop(0, n_chunks, body, 0)

    return k(table, idx_flat)


def kernel(x, x_lengths, x_max_length, batch_size, word_embeddings):
    bsz, seq = x.shape
    B = bsz * seq
    b_per_w = B // NW
    n_chunks = b_per_w // CHUNK
    idx_flat = x.reshape(B).astype(jnp.int32)
    out = _sc_embedding_gather(word_embeddings, idx_flat, b_per_w, n_chunks)
    return out.reshape(bsz, seq, D)


# speedup vs baseline: 4.6717x; 1.1448x over previous
import functools

import jax
import jax.numpy as jnp
from jax import lax
from jax.experimental import pallas as pl
from jax.experimental.pallas import tpu as pltpu
from jax.experimental.pallas import tpu_sc as plsc

D = 64
NW = 32
CHUNK = 400
NBUF = 4
AHEAD = 2


@functools.partial(jax.jit, static_argnums=(2, 3))
def _sc_embedding_gather(table, idx_flat, b_per_w, n_chunks):
    B = b_per_w * NW
    mesh = plsc.VectorSubcoreMesh(core_axis_name="c", subcore_axis_name="s")

    @functools.partial(
        pl.kernel,
        mesh=mesh,
        compiler_params=pltpu.CompilerParams(use_tc_tiling_on_sc=False),
        out_type=jax.ShapeDtypeStruct((B, D), jnp.float32),
        scratch_types=[
            pltpu.VMEM((b_per_w,), jnp.int32),
            [pltpu.VMEM((CHUNK, D), jnp.float32) for _ in range(NBUF)],
            [pltpu.SemaphoreType.DMA for _ in range(NBUF)],
            [pltpu.SemaphoreType.DMA for _ in range(NBUF)],
        ],
    )
    def k(table_hbm, idx_hbm, out_hbm, idx_v, rows, sem_in, sem_out):
        wid = lax.axis_index("s") * 2 + lax.axis_index("c")
        base = wid * b_per_w
        pltpu.sync_copy(idx_hbm.at[pl.ds(base, b_per_w)], idx_v)

        g_h = [None] * n_chunks
        w_h = [None] * n_chunks

        def fire_gather(j):
            b = j % NBUF
            g_h[j] = pltpu.async_copy(
                table_hbm.at[idx_v.at[pl.ds(j * CHUNK, CHUNK)]], rows[b], sem_in[b]
            )

        for j in range(AHEAD):
            fire_gather(j)
        for j in range(n_chunks):
            b = j % NBUF
            jn = j + AHEAD
            if jn < n_chunks:
                if jn >= NBUF:
                    w_h[jn - NBUF].wait()
                fire_gather(jn)
            g_h[j].wait()
            w_h[j] = pltpu.async_copy(
                rows[b], out_hbm.at[pl.ds(base + j * CHUNK, CHUNK)], sem_out[b]
            )
        for j in range(n_chunks - NBUF, n_chunks):
            w_h[j].wait()

    return k(table, idx_flat)


def kernel(x, x_lengths, x_max_length, batch_size, word_embeddings):
    bsz, seq = x.shape
    B = bsz * seq
    b_per_w = B // NW
    n_chunks = b_per_w // CHUNK
    idx_flat = x.reshape(B).astype(jnp.int32)
    out = _sc_embedding_gather(word_embeddings, idx_flat, b_per_w, n_chunks)
    return out.reshape(bsz, seq, D)
